# baseline (speedup 1.0000x reference)
import functools

import jax
import jax.numpy as jnp
from jax import lax
from jax.experimental import pallas as pl
from jax.experimental.pallas import tpu as pltpu
from jax.experimental.pallas import tpu_sc as plsc

NC = 2
NS = 16
NW = NC * NS
LW = 128
KCH = 4
KCS = 2


def _deg_kernel(npad, rows_pw):
  @functools.partial(
      pl.kernel,
      out_type=jax.ShapeDtypeStruct((NC, npad, 128), jnp.float32),
      mesh=plsc.VectorSubcoreMesh(core_axis_name="c", subcore_axis_name="s"),
      scratch_types=[
          pltpu.VMEM((rows_pw, LW), jnp.int32),
          pltpu.VMEM((LW, 128), jnp.float32),
          pltpu.VMEM_SHARED((npad, 128), jnp.float32),
          pltpu.SemaphoreType.DMA,
          pltpu.SemaphoreType.DMA,
      ],
  )
  def deg(dst_hbm, ones_hbm, zeros_hbm, out_hbm, didx, ones_v, acc,
          semA, semB):
    c = lax.axis_index("c")
    s = lax.axis_index("s")
    wid = c * NS + s
    rps = npad // NS
    pltpu.sync_copy(zeros_hbm.at[pl.ds(s * rps, rps)],
                    acc.at[pl.ds(s * rps, rps)])
    pltpu.sync_copy(ones_hbm, ones_v)
    pltpu.sync_copy(dst_hbm.at[pl.ds(wid * rows_pw, rows_pw)], didx)
    plsc.subcore_barrier()

    def pair(k2, carry):
      k = 2 * k2
      pltpu.async_copy(ones_v, acc.at[didx.at[k]], semA, add=True)
      pltpu.async_copy(ones_v, acc.at[didx.at[k + 1]], semB, add=True)
      pltpu.make_async_copy(ones_v, acc.at[didx.at[k]], semA).wait()
      pltpu.make_async_copy(ones_v, acc.at[didx.at[k + 1]], semB).wait()
      return carry

    lax.fori_loop(0, rows_pw // 2, pair, 0)
    plsc.subcore_barrier()
    pltpu.sync_copy(acc.at[pl.ds(s * rps, rps)],
                    out_hbm.at[c, pl.ds(s * rps, rps)])

  return deg


def _scatter_kernel(n, npad, h, rows_pw):
  half = 40 if rows_pw % 40 == 0 else rows_pw

  @functools.partial(
      pl.kernel,
      out_type=jax.ShapeDtypeStruct((NC, npad, h), jnp.float32),
      mesh=plsc.VectorSubcoreMesh(core_axis_name="c", subcore_axis_name="s"),
      scratch_types=[
          pltpu.VMEM((half, LW), jnp.int32),
          pltpu.VMEM((half, LW), jnp.int32),
          pltpu.VMEM((2, LW, h), jnp.float32),
          pltpu.VMEM_SHARED((npad, h), jnp.float32),
          pltpu.SemaphoreType.DMA,
          pltpu.SemaphoreType.DMA,
          pltpu.SemaphoreType.DMA,
          pltpu.SemaphoreType.DMA,
      ],
  )
  def scat(p_hbm, src_hbm, dst_hbm, zeros_hbm, out_hbm,
           sidx, didx, rows, acc, sem0, sem1, sems0, sems1):
    c = lax.axis_index("c")
    s = lax.axis_index("s")
    wid = c * NS + s
    rps = npad // NS
    pltpu.sync_copy(zeros_hbm.at[pl.ds(s * rps, rps)],
                    acc.at[pl.ds(s * rps, rps)])
    plsc.subcore_barrier()

    for blk in range(rows_pw // half):
      base = wid * rows_pw + blk * half
      pltpu.sync_copy(src_hbm.at[pl.ds(base, half)], sidx)
      pltpu.sync_copy(dst_hbm.at[pl.ds(base, half)], didx)
      pltpu.async_copy(p_hbm.at[sidx.at[0]], rows.at[0], sem0)
      pltpu.async_copy(p_hbm.at[sidx.at[1]], rows.at[1], sem1)

      def pair(k2, carry):
        k = 2 * k2
        pltpu.make_async_copy(p_hbm.at[sidx.at[k]], rows.at[0], sem0).wait()
        pltpu.async_copy(rows.at[0], acc.at[didx.at[k]], sems0, add=True)
        pltpu.make_async_copy(p_hbm.at[sidx.at[k + 1]], rows.at[1],
                              sem1).wait()
        pltpu.async_copy(rows.at[1], acc.at[didx.at[k + 1]], sems1, add=True)

        @pl.when(k + 2 < half)
        def _():
          pltpu.make_async_copy(rows.at[0], acc.at[didx.at[k]],
                                sems0).wait()
          pltpu.async_copy(p_hbm.at[sidx.at[k + 2]], rows.at[0], sem0)

        @pl.when(k + 3 < half)
        def _():
          pltpu.make_async_copy(rows.at[1], acc.at[didx.at[k + 1]],
                                sems1).wait()
          pltpu.async_copy(p_hbm.at[sidx.at[k + 3]], rows.at[1], sem1)

        return carry

      lax.fori_loop(0, half // 2, pair, 0)
      pltpu.make_async_copy(rows.at[0], acc.at[didx.at[half - 2]],
                            sems0).wait()
      pltpu.make_async_copy(rows.at[1], acc.at[didx.at[half - 1]],
                            sems1).wait()

    plsc.subcore_barrier()
    pltpu.sync_copy(acc.at[pl.ds(s * rps, rps)],
                    out_hbm.at[c, pl.ds(s * rps, rps)])

  return scat


def _prep_body(x_ref, wp_ref, w1_ref, dacc_ref, id_ref, p1_ref, dinv_ref):
  n = x_ref.shape[0]
  x = x_ref[...]
  deg = dacc_ref[0, 0:n, 0:1] + dacc_ref[1, 0:n, 0:1] + 1.0
  dinv = 1.0 / jnp.sqrt(deg)
  id_ref[...] = jnp.dot(x, wp_ref[...], preferred_element_type=jnp.float32)
  p1_ref[...] = jnp.dot(x, w1_ref[...],
                        preferred_element_type=jnp.float32) * dinv
  dinv_ref[...] = dinv


def _combine_body(s_ref, p_ref, dinv_ref, b_ref, g_ref, be_ref,
                  idn_ref, w_ref, h_ref, pn_ref):
  n = p_ref.shape[0]
  dinv = dinv_ref[...]
  z = dinv * (s_ref[0, 0:n, :] + s_ref[1, 0:n, :] + p_ref[...]) + b_ref[...]
  mu = jnp.mean(z, axis=0, keepdims=True)
  zc = z - mu
  var = jnp.mean(zc * zc, axis=0, keepdims=True)
  hb = g_ref[...] * zc / jnp.sqrt(var + 1e-5) + be_ref[...]
  hh = jnp.maximum(hb, 0.0) + idn_ref[...]
  h_ref[...] = hh
  pn_ref[...] = jnp.dot(hh, w_ref[...],
                        preferred_element_type=jnp.float32) * dinv


def _final_body(s_ref, p_ref, dinv_ref, b_ref, g_ref, be_ref,
                idn_ref, batch_ref, linw_ref, linb_ref, out_ref):
  n = p_ref.shape[0]
  g_segs = out_ref.shape[0]
  dinv = dinv_ref[...]
  z = dinv * (s_ref[0, 0:n, :] + s_ref[1, 0:n, :] + p_ref[...]) + b_ref[...]
  mu = jnp.mean(z, axis=0, keepdims=True)
  zc = z - mu
  var = jnp.mean(zc * zc, axis=0, keepdims=True)
  hb = g_ref[...] * zc / jnp.sqrt(var + 1e-5) + be_ref[...]
  hh = jnp.maximum(hb, 0.0) + idn_ref[...]
  seg_ids = lax.broadcasted_iota(jnp.int32, (n, g_segs), 1)
  mask = (batch_ref[...] == seg_ids).astype(jnp.float32)
  sums = lax.dot_general(mask, hh, (((0,), (0,)), ((), ())),
                         preferred_element_type=jnp.float32,
                         precision=lax.Precision.HIGHEST)
  counts = lax.dot_general(mask, jnp.ones((n, 1), jnp.float32),
                           (((0,), (0,)), ((), ())),
                           preferred_element_type=jnp.float32,
                           precision=lax.Precision.HIGHEST)
  pooled = sums / jnp.maximum(counts, 1.0)
  out_ref[...] = jnp.dot(pooled, linw_ref[...],
                         preferred_element_type=jnp.float32) + linb_ref[...]


def kernel(x, edge_index, batch, Wp, W1, b1, g1, be1, W2, b2, g2, be2,
           W3, b3, g3, be3, W4, b4, g4, be4, W5, b5, g5, be5, linW, linb):
  n, f = x.shape
  h = W1.shape[1]
  e = edge_index.shape[1]
  g_segs = 64

  per_iter = NW * KCH * LW
  rows_pw = ((e + per_iter - 1) // per_iter) * KCH
  epad = NW * rows_pw * LW
  npad = ((n + 128) // 128) * 128
  src = edge_index[0]
  dst = edge_index[1]
  if epad != e:
    pad = epad - e
    src = jnp.concatenate([src, jnp.zeros((pad,), src.dtype)])
    trash = n + jnp.arange(pad, dtype=dst.dtype) % (npad - n)
    dst = jnp.concatenate([dst, trash])
  src2d = src.reshape(-1, LW)
  dst2d = dst.reshape(-1, LW)
  c_edges = KCH * LW

  zeros_h = jnp.zeros((npad, h), jnp.float32)
  ones128 = jnp.ones((LW, 128), jnp.float32)

  deg_call = _deg_kernel(npad, rows_pw)
  scat_call = _scatter_kernel(n, npad, h, rows_pw)

  b1r, g1r, be1r = b1.reshape(1, h), g1.reshape(1, h), be1.reshape(1, h)
  bgbe = [(b1r, g1r, be1r),
          (b2.reshape(1, h), g2.reshape(1, h), be2.reshape(1, h)),
          (b3.reshape(1, h), g3.reshape(1, h), be3.reshape(1, h)),
          (b4.reshape(1, h), g4.reshape(1, h), be4.reshape(1, h)),
          (b5.reshape(1, h), g5.reshape(1, h), be5.reshape(1, h))]
  w_next = [W2, W3, W4, W5]

  dacc = deg_call(dst2d, ones128, zeros_h)

  f32 = jnp.float32
  ident, p, dinv = pl.pallas_call(
      _prep_body,
      out_shape=[jax.ShapeDtypeStruct((n, h), f32),
                 jax.ShapeDtypeStruct((n, h), f32),
                 jax.ShapeDtypeStruct((n, 1), f32)],
  )(x, Wp, W1, dacc)

  idn = ident
  for i in range(4):
    s_part = scat_call(p, src2d, dst2d, zeros_h)
    bi, gi, bei = bgbe[i]
    idn, p = pl.pallas_call(
        _combine_body,
        out_shape=[jax.ShapeDtypeStruct((n, h), f32),
                   jax.ShapeDtypeStruct((n, h), f32)],
    )(s_part, p, dinv, bi, gi, bei, idn, w_next[i])

  s_part = scat_call(p, src2d, dst2d, zeros_h)
  b5r, g5r, be5r = bgbe[4]
  out = pl.pallas_call(
      _final_body,
      out_shape=jax.ShapeDtypeStruct((g_segs, 1), f32),
  )(s_part, p, dinv, b5r, g5r, be5r, idn, batch.reshape(n, 1), linW,
    linb.reshape(1, 1))
  return out


# speedup vs baseline: 1.0387x; 1.0387x over previous
import functools

import jax
import jax.numpy as jnp
from jax import lax
from jax.experimental import pallas as pl
from jax.experimental.pallas import tpu as pltpu
from jax.experimental.pallas import tpu_sc as plsc

NC = 2
NS = 16
NW = NC * NS
LW = 128
KCH = 4
KCS = 2


def _deg_kernel(npad, rows_pw):
  @functools.partial(
      pl.kernel,
      out_type=jax.ShapeDtypeStruct((NC, npad, 128), jnp.float32),
      mesh=plsc.VectorSubcoreMesh(core_axis_name="c", subcore_axis_name="s"),
      scratch_types=[
          pltpu.VMEM((rows_pw, LW), jnp.int32),
          pltpu.VMEM((LW, 128), jnp.float32),
          pltpu.VMEM_SHARED((npad, 128), jnp.float32),
          pltpu.SemaphoreType.DMA,
          pltpu.SemaphoreType.DMA,
      ],
  )
  def deg(dst_hbm, ones_hbm, zeros_hbm, out_hbm, didx, ones_v, acc,
          semA, semB):
    c = lax.axis_index("c")
    s = lax.axis_index("s")
    wid = c * NS + s
    rps = npad // NS
    pltpu.sync_copy(zeros_hbm.at[pl.ds(s * rps, rps)],
                    acc.at[pl.ds(s * rps, rps)])
    pltpu.sync_copy(ones_hbm, ones_v)
    pltpu.sync_copy(dst_hbm.at[pl.ds(wid * rows_pw, rows_pw)], didx)
    plsc.subcore_barrier()

    def pair(k2, carry):
      k = 2 * k2
      pltpu.async_copy(ones_v, acc.at[didx.at[k]], semA, add=True)
      pltpu.async_copy(ones_v, acc.at[didx.at[k + 1]], semB, add=True)
      pltpu.make_async_copy(ones_v, acc.at[didx.at[k]], semA).wait()
      pltpu.make_async_copy(ones_v, acc.at[didx.at[k + 1]], semB).wait()
      return carry

    lax.fori_loop(0, rows_pw // 2, pair, 0)
    plsc.subcore_barrier()
    pltpu.sync_copy(acc.at[pl.ds(s * rps, rps)],
                    out_hbm.at[c, pl.ds(s * rps, rps)])

  return deg


def _scatter_kernel(n, npad, h, rows_pw):
  half = 40 if rows_pw % 40 == 0 else rows_pw

  @functools.partial(
      pl.kernel,
      out_type=jax.ShapeDtypeStruct((NC, npad, h), jnp.float32),
      mesh=plsc.VectorSubcoreMesh(core_axis_name="c", subcore_axis_name="s"),
      scratch_types=[
          pltpu.VMEM((half, LW), jnp.int32),
          pltpu.VMEM((half, LW), jnp.int32),
          pltpu.VMEM((2, LW, h), jnp.float32),
          pltpu.VMEM_SHARED((npad, h), jnp.float32),
          pltpu.SemaphoreType.DMA,
          pltpu.SemaphoreType.DMA,
      ],
  )
  def scat(p_hbm, src_hbm, dst_hbm, zeros_hbm, out_hbm,
           sidx, didx, rows, acc, sem0, sem1):
    c = lax.axis_index("c")
    s = lax.axis_index("s")
    wid = c * NS + s
    rps = npad // NS
    pltpu.sync_copy(zeros_hbm.at[pl.ds(s * rps, rps)],
                    acc.at[pl.ds(s * rps, rps)])
    plsc.subcore_barrier()

    for blk in range(rows_pw // half):
      base = wid * rows_pw + blk * half
      pltpu.sync_copy(src_hbm.at[pl.ds(base, half)], sidx)
      pltpu.sync_copy(dst_hbm.at[pl.ds(base, half)], didx)
      pltpu.async_copy(p_hbm.at[sidx.at[0]], rows.at[0], sem0)

      def pair(k2, carry):
        k = 2 * k2
        pltpu.async_copy(p_hbm.at[sidx.at[k + 1]], rows.at[1], sem1)
        pltpu.make_async_copy(p_hbm.at[sidx.at[k]], rows.at[0], sem0).wait()
        pltpu.sync_copy(rows.at[0], acc.at[didx.at[k]], add=True)

        @pl.when(k + 2 < half)
        def _():
          pltpu.async_copy(p_hbm.at[sidx.at[k + 2]], rows.at[0], sem0)

        pltpu.make_async_copy(p_hbm.at[sidx.at[k + 1]], rows.at[1],
                              sem1).wait()
        pltpu.sync_copy(rows.at[1], acc.at[didx.at[k + 1]], add=True)
        return carry

      lax.fori_loop(0, half // 2, pair, 0)

    plsc.subcore_barrier()
    pltpu.sync_copy(acc.at[pl.ds(s * rps, rps)],
                    out_hbm.at[c, pl.ds(s * rps, rps)])

  return scat


def _prep_body(x_ref, wp_ref, w1_ref, dacc_ref, id_ref, p1_ref, dinv_ref):
  n = x_ref.shape[0]
  x = x_ref[...]
  deg = dacc_ref[0, 0:n, 0:1] + dacc_ref[1, 0:n, 0:1] + 1.0
  dinv = 1.0 / jnp.sqrt(deg)
  id_ref[...] = jnp.dot(x, wp_ref[...], preferred_element_type=jnp.float32)
  p1_ref[...] = jnp.dot(x, w1_ref[...],
                        preferred_element_type=jnp.float32) * dinv
  dinv_ref[...] = dinv


def _combine_body(s_ref, p_ref, dinv_ref, b_ref, g_ref, be_ref,
                  idn_ref, w_ref, h_ref, pn_ref):
  n = p_ref.shape[0]
  dinv = dinv_ref[...]
  z = dinv * (s_ref[0, 0:n, :] + s_ref[1, 0:n, :] + p_ref[...]) + b_ref[...]
  mu = jnp.mean(z, axis=0, keepdims=True)
  zc = z - mu
  var = jnp.mean(zc * zc, axis=0, keepdims=True)
  hb = g_ref[...] * zc / jnp.sqrt(var + 1e-5) + be_ref[...]
  hh = jnp.maximum(hb, 0.0) + idn_ref[...]
  h_ref[...] = hh
  pn_ref[...] = jnp.dot(hh, w_ref[...],
                        preferred_element_type=jnp.float32) * dinv


def _final_body(s_ref, p_ref, dinv_ref, b_ref, g_ref, be_ref,
                idn_ref, batch_ref, linw_ref, linb_ref, out_ref):
  n = p_ref.shape[0]
  g_segs = out_ref.shape[0]
  dinv = dinv_ref[...]
  z = dinv * (s_ref[0, 0:n, :] + s_ref[1, 0:n, :] + p_ref[...]) + b_ref[...]
  mu = jnp.mean(z, axis=0, keepdims=True)
  zc = z - mu
  var = jnp.mean(zc * zc, axis=0, keepdims=True)
  hb = g_ref[...] * zc / jnp.sqrt(var + 1e-5) + be_ref[...]
  hh = jnp.maximum(hb, 0.0) + idn_ref[...]
  seg_ids = lax.broadcasted_iota(jnp.int32, (n, g_segs), 1)
  mask = (batch_ref[...] == seg_ids).astype(jnp.float32)
  sums = lax.dot_general(mask, hh, (((0,), (0,)), ((), ())),
                         preferred_element_type=jnp.float32,
                         precision=lax.Precision.HIGHEST)
  counts = lax.dot_general(mask, jnp.ones((n, 1), jnp.float32),
                           (((0,), (0,)), ((), ())),
                           preferred_element_type=jnp.float32,
                           precision=lax.Precision.HIGHEST)
  pooled = sums / jnp.maximum(counts, 1.0)
  out_ref[...] = jnp.dot(pooled, linw_ref[...],
                         preferred_element_type=jnp.float32) + linb_ref[...]


def kernel(x, edge_index, batch, Wp, W1, b1, g1, be1, W2, b2, g2, be2,
           W3, b3, g3, be3, W4, b4, g4, be4, W5, b5, g5, be5, linW, linb):
  n, f = x.shape
  h = W1.shape[1]
  e = edge_index.shape[1]
  g_segs = 64

  per_iter = NW * KCH * LW
  rows_pw = ((e + per_iter - 1) // per_iter) * KCH
  epad = NW * rows_pw * LW
  npad = ((n + 128) // 128) * 128
  src = edge_index[0]
  dst = edge_index[1]
  if epad != e:
    pad = epad - e
    src = jnp.concatenate([src, jnp.zeros((pad,), src.dtype)])
    trash = n + jnp.arange(pad, dtype=dst.dtype) % (npad - n)
    dst = jnp.concatenate([dst, trash])
  src2d = src.reshape(-1, LW)
  dst2d = dst.reshape(-1, LW)
  c_edges = KCH * LW

  zeros_h = jnp.zeros((npad, h), jnp.float32)
  ones128 = jnp.ones((LW, 128), jnp.float32)

  deg_call = _deg_kernel(npad, rows_pw)
  scat_call = _scatter_kernel(n, npad, h, rows_pw)

  b1r, g1r, be1r = b1.reshape(1, h), g1.reshape(1, h), be1.reshape(1, h)
  bgbe = [(b1r, g1r, be1r),
          (b2.reshape(1, h), g2.reshape(1, h), be2.reshape(1, h)),
          (b3.reshape(1, h), g3.reshape(1, h), be3.reshape(1, h)),
          (b4.reshape(1, h), g4.reshape(1, h), be4.reshape(1, h)),
          (b5.reshape(1, h), g5.reshape(1, h), be5.reshape(1, h))]
  w_next = [W2, W3, W4, W5]

  dacc = deg_call(dst2d, ones128, zeros_h)

  f32 = jnp.float32
  ident, p, dinv = pl.pallas_call(
      _prep_body,
      out_shape=[jax.ShapeDtypeStruct((n, h), f32),
                 jax.ShapeDtypeStruct((n, h), f32),
                 jax.ShapeDtypeStruct((n, 1), f32)],
  )(x, Wp, W1, dacc)

  idn = ident
  for i in range(4):
    s_part = scat_call(p, src2d, dst2d, zeros_h)
    bi, gi, bei = bgbe[i]
    idn, p = pl.pallas_call(
        _combine_body,
        out_shape=[jax.ShapeDtypeStruct((n, h), f32),
                   jax.ShapeDtypeStruct((n, h), f32)],
    )(s_part, p, dinv, bi, gi, bei, idn, w_next[i])

  s_part = scat_call(p, src2d, dst2d, zeros_h)
  b5r, g5r, be5r = bgbe[4]
  out = pl.pallas_call(
      _final_body,
      out_shape=jax.ShapeDtypeStruct((g_segs, 1), f32),
  )(s_part, p, dinv, b5r, g5r, be5r, idn, batch.reshape(n, 1), linW,
    linb.reshape(1, 1))
  return out
